# baseline (speedup 1.0000x reference)
import functools

import jax
import jax.numpy as jnp
from jax.experimental import pallas as pl
from jax.experimental.pallas import tpu as pltpu

NUM_KV = 8
BEAM = 5
VOCAB = 100000
LANES = 128
VROWS = 782
VPAD = VROWS * LANES
KVROWS = 16 * 2048 * 64 // LANES
BR = 2048
NI = KVROWS // BR


def _copy_body(*refs):
    ins = refs[:NUM_KV]
    outs = refs[NUM_KV:]
    for k in range(NUM_KV):
        outs[k][0] = ins[k][...]


def _topk_body(logits_ref, rp_ref, pv_ref, rp_out_ref, idx_ref, prob_ref):
    x = logits_ref[...]
    rows = jax.lax.broadcasted_iota(jnp.int32, (VROWS, LANES), 0)
    cols = jax.lax.broadcasted_iota(jnp.int32, (VROWS, LANES), 1)
    gidx = rows * LANES + cols
    neg = jnp.float32(-jnp.inf)
    big = jnp.int32(2**30)

    vals = []
    idxs = []
    cur = x
    for _ in range(BEAM):
        m = jnp.max(cur)
        i = jnp.min(jnp.where(cur == m, gidx, big))
        vals.append(m)
        idxs.append(i)
        cur = jnp.where(gidx == i, neg, cur)

    m0 = vals[0]
    s = jnp.sum(jnp.exp(x - m0))
    logz = m0 + jnp.log(s)

    pv = pv_ref[0]

    rp = rp_ref[...]
    vcols = jax.lax.broadcasted_iota(jnp.int32, (BEAM, VOCAB), 1)
    hit = functools.reduce(
        jnp.logical_or, [vcols == idxs[k] for k in range(BEAM)]
    )
    rp_out_ref[...] = jnp.where(hit, rp * pv, rp)

    r8 = jax.lax.broadcasted_iota(jnp.int32, (8, LANES), 0)
    iacc = jnp.zeros((8, LANES), jnp.int32)
    pacc = jnp.zeros((8, LANES), jnp.float32)
    for k in range(BEAM):
        iacc = jnp.where(r8 == k, idxs[k], iacc)
        pacc = jnp.where(r8 == k, vals[k] - logz, pacc)
    idx_ref[...] = iacc
    prob_ref[...] = pacc


def kernel(kv_0, kv_1, kv_2, kv_3, kv_4, kv_5, kv_6, kv_7,
           save_id, repeat_penality, logits, penality_value, beam_size):
    kvs = [kv_0, kv_1, kv_2, kv_3, kv_4, kv_5, kv_6, kv_7]
    kv_shape = kvs[0].shape

    kv2d = [kv.reshape(KVROWS, LANES) for kv in kvs]
    rep = pl.pallas_call(
        _copy_body,
        grid=(NI, BEAM),
        in_specs=[pl.BlockSpec((BR, LANES), lambda i, j: (i, 0))] * NUM_KV,
        out_specs=[pl.BlockSpec((1, BR, LANES), lambda i, j: (j, i, 0))] * NUM_KV,
        out_shape=[jax.ShapeDtypeStruct((BEAM, KVROWS, LANES), jnp.float32)] * NUM_KV,
    )(*kv2d)
    saved = [o.reshape((BEAM,) + kv_shape[1:]) for o in rep]

    lpad = jnp.pad(logits, ((0, 0), (0, VPAD - VOCAB)),
                   constant_values=-jnp.inf).reshape(VROWS, LANES)
    pv2 = penality_value.reshape(1)
    rp_out, idx8, prob8 = pl.pallas_call(
        _topk_body,
        in_specs=[
            pl.BlockSpec(memory_space=pltpu.VMEM),
            pl.BlockSpec(memory_space=pltpu.VMEM),
            pl.BlockSpec(memory_space=pltpu.SMEM),
        ],
        out_specs=[
            pl.BlockSpec(memory_space=pltpu.VMEM),
            pl.BlockSpec(memory_space=pltpu.VMEM),
            pl.BlockSpec(memory_space=pltpu.VMEM),
        ],
        out_shape=[
            jax.ShapeDtypeStruct((BEAM, VOCAB), jnp.float32),
            jax.ShapeDtypeStruct((8, LANES), jnp.int32),
            jax.ShapeDtypeStruct((8, LANES), jnp.float32),
        ],
    )(lpad, repeat_penality, pv2)

    tbi = idx8[:BEAM, :1]
    save_id_out = jnp.concatenate([save_id, tbi], axis=-1)
    top_prob = prob8[:BEAM, :1]
    batch_indices = (jnp.arange(BEAM, dtype=jnp.int32)
                     + (jnp.asarray(beam_size, jnp.int32) - jnp.int32(BEAM)))
    max_logits_idx = tbi[0]

    return (*saved, save_id_out, rp_out, top_prob, batch_indices, tbi,
            max_logits_idx)


# speedup vs baseline: 1.1440x; 1.1440x over previous
import functools

import jax
import jax.numpy as jnp
from jax.experimental import pallas as pl
from jax.experimental.pallas import tpu as pltpu

NUM_KV = 8
BEAM = 5
VOCAB = 100000
LANES = 128
VROWS = 782
VPAD = VROWS * LANES
NHEADS = 16
NI = NHEADS


def _copy_body(*refs):
    ins = refs[:NUM_KV]
    outs = refs[NUM_KV:]
    for k in range(NUM_KV):
        outs[k][...] = ins[k][...]


def _topk_body(logits_ref, rp_ref, pv_ref, rp_out_ref, idx_ref, prob_ref):
    x = logits_ref[...]
    rows = jax.lax.broadcasted_iota(jnp.int32, (VROWS, LANES), 0)
    cols = jax.lax.broadcasted_iota(jnp.int32, (VROWS, LANES), 1)
    gidx = rows * LANES + cols
    neg = jnp.float32(-jnp.inf)
    big = jnp.int32(2**30)

    vals = []
    idxs = []
    cur = x
    for _ in range(BEAM):
        m = jnp.max(cur)
        i = jnp.min(jnp.where(cur == m, gidx, big))
        vals.append(m)
        idxs.append(i)
        cur = jnp.where(gidx == i, neg, cur)

    m0 = vals[0]
    s = jnp.sum(jnp.exp(x - m0))
    logz = m0 + jnp.log(s)

    pv = pv_ref[0]

    rp = rp_ref[...]
    vcols = jax.lax.broadcasted_iota(jnp.int32, (BEAM, VOCAB), 1)
    hit = functools.reduce(
        jnp.logical_or, [vcols == idxs[k] for k in range(BEAM)]
    )
    rp_out_ref[...] = jnp.where(hit, rp * pv, rp)

    r8 = jax.lax.broadcasted_iota(jnp.int32, (8, LANES), 0)
    iacc = jnp.zeros((8, LANES), jnp.int32)
    pacc = jnp.zeros((8, LANES), jnp.float32)
    for k in range(BEAM):
        iacc = jnp.where(r8 == k, idxs[k], iacc)
        pacc = jnp.where(r8 == k, vals[k] - logz, pacc)
    idx_ref[...] = iacc
    prob_ref[...] = pacc


def kernel(kv_0, kv_1, kv_2, kv_3, kv_4, kv_5, kv_6, kv_7,
           save_id, repeat_penality, logits, penality_value, beam_size):
    kvs = [kv_0, kv_1, kv_2, kv_3, kv_4, kv_5, kv_6, kv_7]
    kv_shape = kvs[0].shape

    blk = (1, 1) + kv_shape[2:]
    saved = pl.pallas_call(
        _copy_body,
        grid=(NI, BEAM),
        in_specs=[pl.BlockSpec(blk, lambda i, j: (0, i, 0, 0))] * NUM_KV,
        out_specs=[pl.BlockSpec(blk, lambda i, j: (j, i, 0, 0))] * NUM_KV,
        out_shape=[jax.ShapeDtypeStruct((BEAM,) + kv_shape[1:], jnp.float32)] * NUM_KV,
    )(*kvs)

    lpad = jnp.pad(logits, ((0, 0), (0, VPAD - VOCAB)),
                   constant_values=-jnp.inf).reshape(VROWS, LANES)
    pv2 = penality_value.reshape(1)
    rp_out, idx8, prob8 = pl.pallas_call(
        _topk_body,
        in_specs=[
            pl.BlockSpec(memory_space=pltpu.VMEM),
            pl.BlockSpec(memory_space=pltpu.VMEM),
            pl.BlockSpec(memory_space=pltpu.SMEM),
        ],
        out_specs=[
            pl.BlockSpec(memory_space=pltpu.VMEM),
            pl.BlockSpec(memory_space=pltpu.VMEM),
            pl.BlockSpec(memory_space=pltpu.VMEM),
        ],
        out_shape=[
            jax.ShapeDtypeStruct((BEAM, VOCAB), jnp.float32),
            jax.ShapeDtypeStruct((8, LANES), jnp.int32),
            jax.ShapeDtypeStruct((8, LANES), jnp.float32),
        ],
    )(lpad, repeat_penality, pv2)

    tbi = idx8[:BEAM, :1]
    save_id_out = jnp.concatenate([save_id, tbi], axis=-1)
    top_prob = prob8[:BEAM, :1]
    batch_indices = (jnp.arange(BEAM, dtype=jnp.int32)
                     + (jnp.asarray(beam_size, jnp.int32) - jnp.int32(BEAM)))
    max_logits_idx = tbi[0]

    return (*saved, save_id_out, rp_out, top_prob, batch_indices, tbi,
            max_logits_idx)


# speedup vs baseline: 1.1838x; 1.0348x over previous
import functools

import jax
import jax.numpy as jnp
from jax.experimental import pallas as pl
from jax.experimental.pallas import tpu as pltpu

NUM_KV = 8
BEAM = 5
VOCAB = 100000
LANES = 128
VROWS = 782
VPAD = VROWS * LANES
CH = 8
NCHUNK = 16 // CH
NSTEP = NUM_KV * NCHUNK


def _fused_body(*refs):
    kv_in = refs[:NUM_KV]
    logits_ref, rp_ref, pv_ref = refs[NUM_KV:NUM_KV + 3]
    kv_out = refs[NUM_KV + 3:2 * NUM_KV + 3]
    rp_out_ref, idx_ref, prob_ref = refs[2 * NUM_KV + 3:2 * NUM_KV + 6]
    buf, lsem, ssem = refs[2 * NUM_KV + 6:]

    def load(t):
        k, h = divmod(t, NCHUNK)
        pltpu.make_async_copy(
            kv_in[k].at[0, pl.ds(h * CH, CH)], buf.at[t % 2], lsem.at[t % 2]
        ).start()

    def load_wait(t):
        k, h = divmod(t, NCHUNK)
        pltpu.make_async_copy(
            kv_in[k].at[0, pl.ds(h * CH, CH)], buf.at[t % 2], lsem.at[t % 2]
        ).wait()

    def store(t):
        k, h = divmod(t, NCHUNK)
        for j in range(BEAM):
            pltpu.make_async_copy(
                buf.at[t % 2], kv_out[k].at[j, pl.ds(h * CH, CH)],
                ssem.at[t % 2]
            ).start()

    def store_wait(t):
        k, h = divmod(t, NCHUNK)
        for j in range(BEAM):
            pltpu.make_async_copy(
                buf.at[t % 2], kv_out[k].at[j, pl.ds(h * CH, CH)],
                ssem.at[t % 2]
            ).wait()

    load(0)
    load(1)

    x = logits_ref[...]
    rows = jax.lax.broadcasted_iota(jnp.int32, (VROWS, LANES), 0)
    cols = jax.lax.broadcasted_iota(jnp.int32, (VROWS, LANES), 1)
    gidx = rows * LANES + cols
    neg = jnp.float32(-jnp.inf)
    big = jnp.int32(2**30)

    vals = []
    idxs = []
    cur = x
    for _ in range(BEAM):
        m = jnp.max(cur)
        i = jnp.min(jnp.where(cur == m, gidx, big))
        vals.append(m)
        idxs.append(i)
        cur = jnp.where(gidx == i, neg, cur)

    m0 = vals[0]
    s = jnp.sum(jnp.exp(x - m0))
    logz = m0 + jnp.log(s)
    pv = pv_ref[0]

    rp = rp_ref[...]
    vcols = jax.lax.broadcasted_iota(jnp.int32, (BEAM, VOCAB), 1)
    hit = functools.reduce(
        jnp.logical_or, [vcols == idxs[k] for k in range(BEAM)]
    )
    rp_out_ref[...] = jnp.where(hit, rp * pv, rp)

    r8 = jax.lax.broadcasted_iota(jnp.int32, (8, LANES), 0)
    iacc = jnp.zeros((8, LANES), jnp.int32)
    pacc = jnp.zeros((8, LANES), jnp.float32)
    for k in range(BEAM):
        iacc = jnp.where(r8 == k, idxs[k], iacc)
        pacc = jnp.where(r8 == k, vals[k] - logz, pacc)
    idx_ref[...] = iacc
    prob_ref[...] = pacc

    for t in range(NSTEP):
        load_wait(t)
        store(t)
        if t + 2 < NSTEP:
            store_wait(t)
            load(t + 2)
    store_wait(NSTEP - 2)
    store_wait(NSTEP - 1)


def kernel(kv_0, kv_1, kv_2, kv_3, kv_4, kv_5, kv_6, kv_7,
           save_id, repeat_penality, logits, penality_value, beam_size):
    kvs = [kv_0, kv_1, kv_2, kv_3, kv_4, kv_5, kv_6, kv_7]
    kv_shape = kvs[0].shape

    lpad = jnp.pad(logits, ((0, 0), (0, VPAD - VOCAB)),
                   constant_values=-jnp.inf).reshape(VROWS, LANES)
    pv1 = penality_value.reshape(1)

    any_spec = pl.BlockSpec(memory_space=pl.ANY)
    vmem_spec = pl.BlockSpec(memory_space=pltpu.VMEM)
    outs = pl.pallas_call(
        _fused_body,
        in_specs=[any_spec] * NUM_KV + [
            vmem_spec, vmem_spec, pl.BlockSpec(memory_space=pltpu.SMEM)],
        out_specs=[any_spec] * NUM_KV + [vmem_spec, vmem_spec, vmem_spec],
        out_shape=(
            [jax.ShapeDtypeStruct((BEAM,) + kv_shape[1:], jnp.float32)] * NUM_KV
            + [jax.ShapeDtypeStruct((BEAM, VOCAB), jnp.float32),
               jax.ShapeDtypeStruct((8, LANES), jnp.int32),
               jax.ShapeDtypeStruct((8, LANES), jnp.float32)]
        ),
        scratch_shapes=[
            pltpu.VMEM((2, CH) + kv_shape[2:], jnp.float32),
            pltpu.SemaphoreType.DMA((2,)),
            pltpu.SemaphoreType.DMA((2,)),
        ],
    )(*kvs, lpad, repeat_penality, pv1)

    saved = outs[:NUM_KV]
    rp_out, idx8, prob8 = outs[NUM_KV:]

    tbi = idx8[:BEAM, :1]
    save_id_out = jnp.concatenate([save_id, tbi], axis=-1)
    top_prob = prob8[:BEAM, :1]
    batch_indices = (jnp.arange(BEAM, dtype=jnp.int32)
                     + (jnp.asarray(beam_size, jnp.int32) - jnp.int32(BEAM)))
    max_logits_idx = tbi[0]

    return (*saved, save_id_out, rp_out, top_prob, batch_indices, tbi,
            max_logits_idx)


# speedup vs baseline: 6.7593x; 5.7099x over previous
import functools

import jax
import jax.numpy as jnp
from jax.experimental import pallas as pl
from jax.experimental.pallas import tpu as pltpu

NUM_KV = 8
BEAM = 5
VOCAB = 100000
LANES = 128
VROWS = 782
VPAD = VROWS * LANES
CH = 8
NCHUNK = 16 // CH
NSTEP = NUM_KV * NCHUNK


def _fused_body(*refs):
    kv_in = refs[:NUM_KV]
    logits_ref, rp_ref, pv_ref = refs[NUM_KV:NUM_KV + 3]
    kv_out = refs[NUM_KV + 3:2 * NUM_KV + 3]
    rp_out_ref, idx_ref, prob_ref = refs[2 * NUM_KV + 3:2 * NUM_KV + 6]
    buf, lsem, ssem = refs[2 * NUM_KV + 6:]

    def load(t):
        k, h = divmod(t, NCHUNK)
        pltpu.make_async_copy(
            kv_in[k].at[0, pl.ds(h * CH, CH)], buf.at[t % 2], lsem.at[t % 2]
        ).start()

    def load_wait(t):
        k, h = divmod(t, NCHUNK)
        pltpu.make_async_copy(
            kv_in[k].at[0, pl.ds(h * CH, CH)], buf.at[t % 2], lsem.at[t % 2]
        ).wait()

    def store(t):
        k, h = divmod(t, NCHUNK)
        for j in range(BEAM):
            pltpu.make_async_copy(
                buf.at[t % 2], kv_out[k].at[j, pl.ds(h * CH, CH)],
                ssem.at[t % 2]
            ).start()

    def store_wait(t):
        k, h = divmod(t, NCHUNK)
        for j in range(BEAM):
            pltpu.make_async_copy(
                buf.at[t % 2], kv_out[k].at[j, pl.ds(h * CH, CH)],
                ssem.at[t % 2]
            ).wait()

    load(0)
    load(1)

    x = logits_ref[...]
    rows = jax.lax.broadcasted_iota(jnp.int32, (VROWS, LANES), 0)
    cols = jax.lax.broadcasted_iota(jnp.int32, (VROWS, LANES), 1)
    gidx = rows * LANES + cols
    neg = jnp.float32(-jnp.inf)
    big = jnp.int32(2**30)

    vals = []
    idxs = []
    cur = x
    for _ in range(BEAM):
        m = jnp.max(cur)
        i = jnp.min(jnp.where(cur == m, gidx, big))
        vals.append(m)
        idxs.append(i)
        cur = jnp.where(gidx == i, neg, cur)

    m0 = vals[0]
    s = jnp.sum(jnp.exp(x - m0))
    logz = m0 + jnp.log(s)
    pv = pv_ref[0]

    rp = rp_ref[...]
    vcols = jax.lax.broadcasted_iota(jnp.int32, (BEAM, VOCAB), 1)
    hit = functools.reduce(
        jnp.logical_or, [vcols == idxs[k] for k in range(BEAM)]
    )
    rp_out_ref[...] = jnp.where(hit, rp * pv, rp)

    r8 = jax.lax.broadcasted_iota(jnp.int32, (8, LANES), 0)
    iacc = jnp.zeros((8, LANES), jnp.int32)
    pacc = jnp.zeros((8, LANES), jnp.float32)
    for k in range(BEAM):
        iacc = jnp.where(r8 == k, idxs[k], iacc)
        pacc = jnp.where(r8 == k, vals[k] - logz, pacc)
    idx_ref[...] = iacc
    prob_ref[...] = pacc

    for t in range(NSTEP):
        load_wait(t)
        store(t)
        if t + 2 < NSTEP:
            store_wait(t)
            load(t + 2)
    store_wait(NSTEP - 2)
    store_wait(NSTEP - 1)


def kernel(kv_0, kv_1, kv_2, kv_3, kv_4, kv_5, kv_6, kv_7,
           save_id, repeat_penality, logits, penality_value, beam_size):
    kvs = [kv_0, kv_1, kv_2, kv_3, kv_4, kv_5, kv_6, kv_7]
    kvs = [jnp.transpose(kv, (0, 1, 3, 2)) for kv in kvs]
    kv_shape = kvs[0].shape

    lpad = jnp.pad(logits, ((0, 0), (0, VPAD - VOCAB)),
                   constant_values=-jnp.inf).reshape(VROWS, LANES)
    pv1 = penality_value.reshape(1)

    any_spec = pl.BlockSpec(memory_space=pl.ANY)
    vmem_spec = pl.BlockSpec(memory_space=pltpu.VMEM)
    outs = pl.pallas_call(
        _fused_body,
        in_specs=[any_spec] * NUM_KV + [
            vmem_spec, vmem_spec, pl.BlockSpec(memory_space=pltpu.SMEM)],
        out_specs=[any_spec] * NUM_KV + [vmem_spec, vmem_spec, vmem_spec],
        out_shape=(
            [jax.ShapeDtypeStruct((BEAM,) + kv_shape[1:], jnp.float32)] * NUM_KV
            + [jax.ShapeDtypeStruct((BEAM, VOCAB), jnp.float32),
               jax.ShapeDtypeStruct((8, LANES), jnp.int32),
               jax.ShapeDtypeStruct((8, LANES), jnp.float32)]
        ),
        scratch_shapes=[
            pltpu.VMEM((2, CH) + kv_shape[2:], jnp.float32),
            pltpu.SemaphoreType.DMA((2,)),
            pltpu.SemaphoreType.DMA((2,)),
        ],
    )(*kvs, lpad, repeat_penality, pv1)

    saved = [jnp.transpose(o, (0, 1, 3, 2)) for o in outs[:NUM_KV]]
    rp_out, idx8, prob8 = outs[NUM_KV:]

    tbi = idx8[:BEAM, :1]
    save_id_out = jnp.concatenate([save_id, tbi], axis=-1)
    top_prob = prob8[:BEAM, :1]
    batch_indices = (jnp.arange(BEAM, dtype=jnp.int32)
                     + (jnp.asarray(beam_size, jnp.int32) - jnp.int32(BEAM)))
    max_logits_idx = tbi[0]

    return (*saved, save_id_out, rp_out, top_prob, batch_indices, tbi,
            max_logits_idx)


# speedup vs baseline: 7.1611x; 1.0594x over previous
import functools

import jax
import jax.numpy as jnp
from jax.experimental import pallas as pl
from jax.experimental.pallas import tpu as pltpu

NUM_KV = 8
BEAM = 5
VOCAB = 100000
LANES = 128
VROWS = 782
VPAD = VROWS * LANES
CH = 16
NCHUNK = 16 // CH
NSTEP = NUM_KV * NCHUNK


def _fused_body(*refs):
    kv_in = refs[:NUM_KV]
    logits_ref, rp_ref, pv_ref = refs[NUM_KV:NUM_KV + 3]
    kv_out = refs[NUM_KV + 3:2 * NUM_KV + 3]
    rp_out_ref, idx_ref, prob_ref = refs[2 * NUM_KV + 3:2 * NUM_KV + 6]
    buf, lsem, ssem = refs[2 * NUM_KV + 6:]

    def load(t):
        k, h = divmod(t, NCHUNK)
        pltpu.make_async_copy(
            kv_in[k].at[0, pl.ds(h * CH, CH)], buf.at[t % 2], lsem.at[t % 2]
        ).start()

    def load_wait(t):
        k, h = divmod(t, NCHUNK)
        pltpu.make_async_copy(
            kv_in[k].at[0, pl.ds(h * CH, CH)], buf.at[t % 2], lsem.at[t % 2]
        ).wait()

    def store(t):
        k, h = divmod(t, NCHUNK)
        for j in range(BEAM):
            pltpu.make_async_copy(
                buf.at[t % 2], kv_out[k].at[j, pl.ds(h * CH, CH)],
                ssem.at[t % 2]
            ).start()

    def store_wait(t):
        k, h = divmod(t, NCHUNK)
        for j in range(BEAM):
            pltpu.make_async_copy(
                buf.at[t % 2], kv_out[k].at[j, pl.ds(h * CH, CH)],
                ssem.at[t % 2]
            ).wait()

    load(0)
    load(1)

    x = logits_ref[...]
    rows = jax.lax.broadcasted_iota(jnp.int32, (VROWS, LANES), 0)
    cols = jax.lax.broadcasted_iota(jnp.int32, (VROWS, LANES), 1)
    gidx = rows * LANES + cols
    neg = jnp.float32(-jnp.inf)
    big = jnp.int32(2**30)

    vals = []
    idxs = []
    cur = x
    for _ in range(BEAM):
        m = jnp.max(cur)
        i = jnp.min(jnp.where(cur == m, gidx, big))
        vals.append(m)
        idxs.append(i)
        cur = jnp.where(gidx == i, neg, cur)

    m0 = vals[0]
    s = jnp.sum(jnp.exp(x - m0))
    logz = m0 + jnp.log(s)
    pv = pv_ref[0]

    rp = rp_ref[...]
    vcols = jax.lax.broadcasted_iota(jnp.int32, (BEAM, VOCAB), 1)
    hit = functools.reduce(
        jnp.logical_or, [vcols == idxs[k] for k in range(BEAM)]
    )
    rp_out_ref[...] = jnp.where(hit, rp * pv, rp)

    r8 = jax.lax.broadcasted_iota(jnp.int32, (8, LANES), 0)
    iacc = jnp.zeros((8, LANES), jnp.int32)
    pacc = jnp.zeros((8, LANES), jnp.float32)
    for k in range(BEAM):
        iacc = jnp.where(r8 == k, idxs[k], iacc)
        pacc = jnp.where(r8 == k, vals[k] - logz, pacc)
    idx_ref[...] = iacc
    prob_ref[...] = pacc

    for t in range(NSTEP):
        load_wait(t)
        store(t)
        if t + 2 < NSTEP:
            store_wait(t)
            load(t + 2)
    store_wait(NSTEP - 2)
    store_wait(NSTEP - 1)


def kernel(kv_0, kv_1, kv_2, kv_3, kv_4, kv_5, kv_6, kv_7,
           save_id, repeat_penality, logits, penality_value, beam_size):
    kvs = [kv_0, kv_1, kv_2, kv_3, kv_4, kv_5, kv_6, kv_7]
    kvs = [jnp.transpose(kv, (0, 1, 3, 2)) for kv in kvs]
    kv_shape = kvs[0].shape

    lpad = jnp.pad(logits, ((0, 0), (0, VPAD - VOCAB)),
                   constant_values=-jnp.inf).reshape(VROWS, LANES)
    pv1 = penality_value.reshape(1)

    any_spec = pl.BlockSpec(memory_space=pl.ANY)
    vmem_spec = pl.BlockSpec(memory_space=pltpu.VMEM)
    outs = pl.pallas_call(
        _fused_body,
        in_specs=[any_spec] * NUM_KV + [
            vmem_spec, vmem_spec, pl.BlockSpec(memory_space=pltpu.SMEM)],
        out_specs=[any_spec] * NUM_KV + [vmem_spec, vmem_spec, vmem_spec],
        out_shape=(
            [jax.ShapeDtypeStruct((BEAM,) + kv_shape[1:], jnp.float32)] * NUM_KV
            + [jax.ShapeDtypeStruct((BEAM, VOCAB), jnp.float32),
               jax.ShapeDtypeStruct((8, LANES), jnp.int32),
               jax.ShapeDtypeStruct((8, LANES), jnp.float32)]
        ),
        scratch_shapes=[
            pltpu.VMEM((2, CH) + kv_shape[2:], jnp.float32),
            pltpu.SemaphoreType.DMA((2,)),
            pltpu.SemaphoreType.DMA((2,)),
        ],
    )(*kvs, lpad, repeat_penality, pv1)

    saved = [jnp.transpose(o, (0, 1, 3, 2)) for o in outs[:NUM_KV]]
    rp_out, idx8, prob8 = outs[NUM_KV:]

    tbi = idx8[:BEAM, :1]
    save_id_out = jnp.concatenate([save_id, tbi], axis=-1)
    top_prob = prob8[:BEAM, :1]
    batch_indices = (jnp.arange(BEAM, dtype=jnp.int32)
                     + (jnp.asarray(beam_size, jnp.int32) - jnp.int32(BEAM)))
    max_logits_idx = tbi[0]

    return (*saved, save_id_out, rp_out, top_prob, batch_indices, tbi,
            max_logits_idx)


# speedup vs baseline: 7.2751x; 1.0159x over previous
import functools

import jax
import jax.numpy as jnp
from jax.experimental import pallas as pl
from jax.experimental.pallas import tpu as pltpu

NUM_KV = 8
BEAM = 5
VOCAB = 100000
LANES = 128
VROWS = 782
VPAD = VROWS * LANES
CH = 16
NCHUNK = 16 // CH
NSTEP = NUM_KV * NCHUNK
NBUF = 3


def _fused_body(*refs):
    kv_in = refs[:NUM_KV]
    logits_ref, rp_ref, pv_ref = refs[NUM_KV:NUM_KV + 3]
    kv_out = refs[NUM_KV + 3:2 * NUM_KV + 3]
    rp_out_ref, idx_ref, prob_ref = refs[2 * NUM_KV + 3:2 * NUM_KV + 6]
    buf, lsem, ssem = refs[2 * NUM_KV + 6:]

    def load(t):
        k, h = divmod(t, NCHUNK)
        pltpu.make_async_copy(
            kv_in[k].at[0, pl.ds(h * CH, CH)], buf.at[t % NBUF],
            lsem.at[t % NBUF]
        ).start()

    def load_wait(t):
        k, h = divmod(t, NCHUNK)
        pltpu.make_async_copy(
            kv_in[k].at[0, pl.ds(h * CH, CH)], buf.at[t % NBUF],
            lsem.at[t % NBUF]
        ).wait()

    def store(t):
        k, h = divmod(t, NCHUNK)
        for j in range(BEAM):
            pltpu.make_async_copy(
                buf.at[t % NBUF], kv_out[k].at[j, pl.ds(h * CH, CH)],
                ssem.at[t % NBUF]
            ).start()

    def store_wait(t):
        k, h = divmod(t, NCHUNK)
        for j in range(BEAM):
            pltpu.make_async_copy(
                buf.at[t % NBUF], kv_out[k].at[j, pl.ds(h * CH, CH)],
                ssem.at[t % NBUF]
            ).wait()

    for t in range(min(NBUF, NSTEP)):
        load(t)

    x = logits_ref[...]
    rows = jax.lax.broadcasted_iota(jnp.int32, (VROWS, LANES), 0)
    cols = jax.lax.broadcasted_iota(jnp.int32, (VROWS, LANES), 1)
    gidx = rows * LANES + cols
    neg = jnp.float32(-jnp.inf)
    big = jnp.int32(2**30)

    vals = []
    idxs = []
    cur = x
    for _ in range(BEAM):
        m = jnp.max(cur)
        i = jnp.min(jnp.where(cur == m, gidx, big))
        vals.append(m)
        idxs.append(i)
        cur = jnp.where(gidx == i, neg, cur)

    m0 = vals[0]
    s = jnp.sum(jnp.exp(x - m0))
    logz = m0 + jnp.log(s)
    pv = pv_ref[0]

    rp = rp_ref[...]
    vcols = jax.lax.broadcasted_iota(jnp.int32, (BEAM, VOCAB), 1)
    hit = functools.reduce(
        jnp.logical_or, [vcols == idxs[k] for k in range(BEAM)]
    )
    rp_out_ref[...] = jnp.where(hit, rp * pv, rp)

    r8 = jax.lax.broadcasted_iota(jnp.int32, (8, LANES), 0)
    iacc = jnp.zeros((8, LANES), jnp.int32)
    pacc = jnp.zeros((8, LANES), jnp.float32)
    for k in range(BEAM):
        iacc = jnp.where(r8 == k, idxs[k], iacc)
        pacc = jnp.where(r8 == k, vals[k] - logz, pacc)
    idx_ref[...] = iacc
    prob_ref[...] = pacc

    for t in range(NSTEP):
        load_wait(t)
        store(t)
        if t + NBUF < NSTEP:
            store_wait(t)
            load(t + NBUF)
    for t in range(max(0, NSTEP - NBUF), NSTEP):
        store_wait(t)


def kernel(kv_0, kv_1, kv_2, kv_3, kv_4, kv_5, kv_6, kv_7,
           save_id, repeat_penality, logits, penality_value, beam_size):
    kvs = [kv_0, kv_1, kv_2, kv_3, kv_4, kv_5, kv_6, kv_7]
    kvs = [jnp.transpose(kv, (0, 1, 3, 2)) for kv in kvs]
    kv_shape = kvs[0].shape

    lpad = jnp.pad(logits, ((0, 0), (0, VPAD - VOCAB)),
                   constant_values=-jnp.inf).reshape(VROWS, LANES)
    pv1 = penality_value.reshape(1)

    any_spec = pl.BlockSpec(memory_space=pl.ANY)
    vmem_spec = pl.BlockSpec(memory_space=pltpu.VMEM)
    outs = pl.pallas_call(
        _fused_body,
        in_specs=[any_spec] * NUM_KV + [
            vmem_spec, vmem_spec, pl.BlockSpec(memory_space=pltpu.SMEM)],
        out_specs=[any_spec] * NUM_KV + [vmem_spec, vmem_spec, vmem_spec],
        out_shape=(
            [jax.ShapeDtypeStruct((BEAM,) + kv_shape[1:], jnp.float32)] * NUM_KV
            + [jax.ShapeDtypeStruct((BEAM, VOCAB), jnp.float32),
               jax.ShapeDtypeStruct((8, LANES), jnp.int32),
               jax.ShapeDtypeStruct((8, LANES), jnp.float32)]
        ),
        scratch_shapes=[
            pltpu.VMEM((NBUF, CH) + kv_shape[2:], jnp.float32),
            pltpu.SemaphoreType.DMA((NBUF,)),
            pltpu.SemaphoreType.DMA((NBUF,)),
        ],
    )(*kvs, lpad, repeat_penality, pv1)

    saved = [jnp.transpose(o, (0, 1, 3, 2)) for o in outs[:NUM_KV]]
    rp_out, idx8, prob8 = outs[NUM_KV:]

    tbi = idx8[:BEAM, :1]
    save_id_out = jnp.concatenate([save_id, tbi], axis=-1)
    top_prob = prob8[:BEAM, :1]
    batch_indices = (jnp.arange(BEAM, dtype=jnp.int32)
                     + (jnp.asarray(beam_size, jnp.int32) - jnp.int32(BEAM)))
    max_logits_idx = tbi[0]

    return (*saved, save_id_out, rp_out, top_prob, batch_indices, tbi,
            max_logits_idx)
